# baseline (speedup 1.0000x reference)
import functools

import jax
import jax.numpy as jnp
from jax import lax
from jax.experimental import pallas as pl
from jax.experimental.pallas import tpu as pltpu
from jax.experimental.pallas import tpu_sc as plsc

N_NODES = 10000
N_PAD = 10240
N_EDGES = 320000
NFEAT = 128
NHID = 64

NC = 2
NS = 16
NW = NC * NS
K = 80
NCHUNK = 125
EPT = NCHUNK * K
E_PAD = NW * EPT
STRIPE = N_PAD // NS
ZK = 80

_mesh = plsc.VectorSubcoreMesh(core_axis_name="c", subcore_axis_name="s")


@functools.partial(
    pl.kernel,
    out_type=(
        jax.ShapeDtypeStruct((NW, N_PAD), jnp.float32),
        jax.ShapeDtypeStruct((NW, N_PAD), jnp.float32),
    ),
    mesh=_mesh,
    compiler_params=pltpu.CompilerParams(needs_layout_passes=False),
    scratch_types=(
        pltpu.VMEM((EPT,), jnp.int32),
        pltpu.VMEM((EPT,), jnp.int32),
        pltpu.VMEM((N_PAD,), jnp.float32),
        pltpu.VMEM((N_PAD,), jnp.float32),
    ),
)
def _deg_kernel(src_hbm, dst_hbm, zeros_hbm, dsrc_hbm, ddst_hbm,
                src_v, dst_v, hsrc_v, hdst_v):
    c = lax.axis_index("c")
    s = lax.axis_index("s")
    wid = c * NS + s
    pltpu.sync_copy(zeros_hbm, hsrc_v)
    pltpu.sync_copy(zeros_hbm, hdst_v)
    pltpu.sync_copy(src_hbm.at[wid], src_v)
    pltpu.sync_copy(dst_hbm.at[wid], dst_v)
    ones = jnp.full((16,), 1.0, jnp.float32)

    def body(i, carry):
        plsc.addupdate_scatter(hsrc_v, [src_v[pl.ds(i * 16, 16)]], ones)
        plsc.addupdate_scatter(hdst_v, [dst_v[pl.ds(i * 16, 16)]], ones)
        return carry

    lax.fori_loop(0, EPT // 16, body, 0)
    pltpu.sync_copy(hsrc_v, dsrc_hbm.at[wid])
    pltpu.sync_copy(hdst_v, ddst_hbm.at[wid])


def _make_agg_kernel(d):
    @functools.partial(
        pl.kernel,
        out_type=jax.ShapeDtypeStruct((NC, N_PAD, d), jnp.float32),
        mesh=_mesh,
        compiler_params=pltpu.CompilerParams(use_tc_tiling_on_sc=False),
        scratch_types=(
            pltpu.VMEM((NCHUNK, K), jnp.int32),
            pltpu.VMEM((NCHUNK, K), jnp.int32),
            pltpu.VMEM((K, d), jnp.float32),
            pltpu.VMEM((K, d), jnp.float32),
            pltpu.VMEM_SHARED((N_PAD, d), jnp.float32),
            pltpu.SemaphoreType.DMA,
            pltpu.SemaphoreType.DMA,
            pltpu.SemaphoreType.DMA,
            pltpu.SemaphoreType.DMA,
        ),
    )
    def agg(h_hbm, src_hbm, dst_hbm, zeros_hbm, out_hbm,
            src_v, dst_v, rows_a, rows_b, acc_sh,
            gsem_a, gsem_b, ssem_a, ssem_b):
        c = lax.axis_index("c")
        s = lax.axis_index("s")
        wid = c * NS + s
        row0 = pl.multiple_of(s * STRIPE, 8)
        pltpu.sync_copy(zeros_hbm, rows_a.at[pl.ds(0, ZK)])
        for j in range(STRIPE // ZK):
            pltpu.sync_copy(rows_a.at[pl.ds(0, ZK)],
                            acc_sh.at[pl.ds(row0 + j * ZK, ZK)])
        pltpu.sync_copy(src_hbm.at[wid], src_v)
        pltpu.sync_copy(dst_hbm.at[wid], dst_v)
        plsc.subcore_barrier()

        def g_start(i, buf, sem):
            pltpu.async_copy(h_hbm.at[src_v.at[i]], buf, sem)

        def g_wait(i, buf, sem):
            pltpu.make_async_copy(h_hbm.at[src_v.at[i]], buf, sem).wait()

        def s_start(i, buf, sem):
            pltpu.async_copy(buf, acc_sh.at[dst_v.at[i]], sem, add=True)

        def s_wait(i, buf, sem):
            pltpu.make_async_copy(buf, acc_sh.at[dst_v.at[i]], sem).wait()

        g_start(0, rows_a, gsem_a)
        g_start(1, rows_b, gsem_b)

        def body(j, carry):
            i0 = 2 * j
            i1 = i0 + 1
            g_wait(i0, rows_a, gsem_a)
            s_start(i0, rows_a, ssem_a)
            g_wait(i1, rows_b, gsem_b)
            s_start(i1, rows_b, ssem_b)
            s_wait(i0, rows_a, ssem_a)
            g_start(i0 + 2, rows_a, gsem_a)
            s_wait(i1, rows_b, ssem_b)

            @pl.when(i1 + 2 < NCHUNK)
            def _():
                g_start(i1 + 2, rows_b, gsem_b)

            return carry

        lax.fori_loop(0, NCHUNK // 2, body, 0)
        g_wait(NCHUNK - 1, rows_a, gsem_a)
        s_start(NCHUNK - 1, rows_a, ssem_a)
        s_wait(NCHUNK - 1, rows_a, ssem_a)
        plsc.subcore_barrier()
        for j in range(STRIPE // ZK):
            pltpu.sync_copy(acc_sh.at[pl.ds(row0 + j * ZK, ZK)],
                            rows_a.at[pl.ds(0, ZK)])
            pltpu.sync_copy(rows_a.at[pl.ds(0, ZK)],
                            out_hbm.at[c, pl.ds(row0 + j * ZK, ZK)])

    return agg


_agg128 = _make_agg_kernel(NFEAT)
_agg64 = _make_agg_kernel(NHID)


_BLK = 1024
_GRID = N_PAD // _BLK


def _prep_body(feat_ref, dsrc_ref, ddst_ref, h1_ref, sout_ref, sin_ref):
    i = pl.program_id(0)
    d_out = jnp.sum(dsrc_ref[:, pl.ds(i * _BLK, _BLK)], axis=0)
    d_in = jnp.sum(ddst_ref[:, pl.ds(i * _BLK, _BLK)], axis=0)
    s_out = lax.rsqrt(jnp.maximum(d_out, 1.0))
    s_in = lax.rsqrt(jnp.maximum(d_in, 1.0))
    h1_ref[...] = feat_ref[...] * s_out[:, None]
    sout_ref[...] = s_out[:, None]
    sin_ref[...] = s_in[:, None]


_prep = pl.pallas_call(
    _prep_body,
    grid=(_GRID,),
    in_specs=[
        pl.BlockSpec((_BLK, NFEAT), lambda i: (i, 0)),
        pl.BlockSpec((NW, N_PAD), lambda i: (0, 0)),
        pl.BlockSpec((NW, N_PAD), lambda i: (0, 0)),
    ],
    out_specs=[
        pl.BlockSpec((_BLK, NFEAT), lambda i: (i, 0)),
        pl.BlockSpec((_BLK, 1), lambda i: (i, 0)),
        pl.BlockSpec((_BLK, 1), lambda i: (i, 0)),
    ],
    out_shape=[
        jax.ShapeDtypeStruct((N_PAD, NFEAT), jnp.float32),
        jax.ShapeDtypeStruct((N_PAD, 1), jnp.float32),
        jax.ShapeDtypeStruct((N_PAD, 1), jnp.float32),
    ],
)


def _mid_body(agg_ref, sin_ref, sout_ref, w1_ref, b1_ref, w2_ref, h2_ref):
    a = (agg_ref[0] + agg_ref[1]) * sin_ref[...]
    x = jnp.maximum(
        jnp.dot(a, w1_ref[...], preferred_element_type=jnp.float32)
        + b1_ref[...], 0.0)
    h2_ref[...] = jnp.dot(x * sout_ref[...], w2_ref[...],
                          preferred_element_type=jnp.float32)


_mid = pl.pallas_call(
    _mid_body,
    grid=(_GRID,),
    in_specs=[
        pl.BlockSpec((NC, _BLK, NFEAT), lambda i: (0, i, 0)),
        pl.BlockSpec((_BLK, 1), lambda i: (i, 0)),
        pl.BlockSpec((_BLK, 1), lambda i: (i, 0)),
        pl.BlockSpec((NFEAT, NFEAT), lambda i: (0, 0)),
        pl.BlockSpec((1, NFEAT), lambda i: (0, 0)),
        pl.BlockSpec((NFEAT, NHID), lambda i: (0, 0)),
    ],
    out_specs=pl.BlockSpec((_BLK, NHID), lambda i: (i, 0)),
    out_shape=jax.ShapeDtypeStruct((N_PAD, NHID), jnp.float32),
)


def _final_body(agg_ref, sin_ref, b2_ref, out_ref):
    z = (agg_ref[0] + agg_ref[1]) * sin_ref[...] + b2_ref[...]
    m = jnp.max(z, axis=1, keepdims=True)
    e = jnp.exp(z - m)
    out_ref[...] = (z - m) - jnp.log(jnp.sum(e, axis=1, keepdims=True))


_final = pl.pallas_call(
    _final_body,
    grid=(_GRID,),
    in_specs=[
        pl.BlockSpec((NC, _BLK, NHID), lambda i: (0, i, 0)),
        pl.BlockSpec((_BLK, 1), lambda i: (i, 0)),
        pl.BlockSpec((1, NHID), lambda i: (0, 0)),
    ],
    out_specs=pl.BlockSpec((_BLK, NHID), lambda i: (i, 0)),
    out_shape=jax.ShapeDtypeStruct((N_NODES, NHID), jnp.float32),
)


def kernel(feat, edge_index, W1, b1, W2, b2):
    src = edge_index[0].astype(jnp.int32)
    dst = edge_index[1].astype(jnp.int32)
    src3 = src.reshape(NW, NCHUNK, K)
    dst3 = dst.reshape(NW, NCHUNK, K)
    zeros_n = jnp.zeros((N_PAD,), jnp.float32)
    zeros128 = jnp.zeros((ZK, NFEAT), jnp.float32)
    zeros64 = jnp.zeros((ZK, NHID), jnp.float32)

    dsrc, ddst = _deg_kernel(src.reshape(NW, EPT), dst.reshape(NW, EPT),
                             zeros_n)
    h1, s_out, s_in = _prep(feat, dsrc, ddst)
    agg1 = _agg128(h1, src3, dst3, zeros128)
    h2 = _mid(agg1, s_in, s_out, W1, b1.reshape(1, NFEAT), W2)
    agg2 = _agg64(h2, src3, dst3, zeros64)
    return _final(agg2, s_in, b2.reshape(1, NHID))


# speedup vs baseline: 1.0996x; 1.0996x over previous
import functools

import jax
import jax.numpy as jnp
from jax import lax
from jax.experimental import pallas as pl
from jax.experimental.pallas import tpu as pltpu
from jax.experimental.pallas import tpu_sc as plsc

N_NODES = 10000
N_PAD = 10240
N_EDGES = 320000
NFEAT = 128
NHID = 64

NC = 2
NS = 16
NW = NC * NS
K = 80
NCHUNK = 125
EPT = NCHUNK * K
E_PAD = NW * EPT
STRIPE = N_PAD // NS
ZK = 80

_mesh = plsc.VectorSubcoreMesh(core_axis_name="c", subcore_axis_name="s")


@functools.partial(
    pl.kernel,
    out_type=(
        jax.ShapeDtypeStruct((NW, N_PAD), jnp.float32),
        jax.ShapeDtypeStruct((NW, N_PAD), jnp.float32),
    ),
    mesh=_mesh,
    compiler_params=pltpu.CompilerParams(needs_layout_passes=False),
    scratch_types=(
        pltpu.VMEM((EPT,), jnp.int32),
        pltpu.VMEM((EPT,), jnp.int32),
        pltpu.VMEM((N_PAD,), jnp.float32),
        pltpu.VMEM((N_PAD,), jnp.float32),
    ),
)
def _deg_kernel(src_hbm, dst_hbm, zeros_hbm, dsrc_hbm, ddst_hbm,
                src_v, dst_v, hsrc_v, hdst_v):
    c = lax.axis_index("c")
    s = lax.axis_index("s")
    wid = c * NS + s
    pltpu.sync_copy(zeros_hbm, hsrc_v)
    pltpu.sync_copy(zeros_hbm, hdst_v)
    pltpu.sync_copy(src_hbm.at[wid], src_v)
    pltpu.sync_copy(dst_hbm.at[wid], dst_v)
    ones = jnp.full((16,), 1.0, jnp.float32)

    def body(i, carry):
        plsc.addupdate_scatter(hsrc_v, [src_v[pl.ds(i * 16, 16)]], ones)
        plsc.addupdate_scatter(hdst_v, [dst_v[pl.ds(i * 16, 16)]], ones)
        return carry

    lax.fori_loop(0, EPT // 16, body, 0)
    pltpu.sync_copy(hsrc_v, dsrc_hbm.at[wid])
    pltpu.sync_copy(hdst_v, ddst_hbm.at[wid])


def _make_agg_kernel(d, nbuf):
    @functools.partial(
        pl.kernel,
        out_type=jax.ShapeDtypeStruct((NC, N_PAD, d), jnp.float32),
        mesh=_mesh,
        compiler_params=pltpu.CompilerParams(use_tc_tiling_on_sc=False),
        scratch_types=(
            pltpu.VMEM((NCHUNK, K), jnp.int32),
            pltpu.VMEM((NCHUNK, K), jnp.int32),
            tuple(pltpu.VMEM((K, d), jnp.float32) for _ in range(nbuf)),
            pltpu.VMEM_SHARED((N_PAD, d), jnp.float32),
            tuple(pltpu.SemaphoreType.DMA for _ in range(2 * nbuf)),
        ),
    )
    def agg(h_hbm, src_hbm, dst_hbm, zeros_hbm, out_hbm,
            src_v, dst_v, rows, acc_sh, sems):
        gsems = sems[:nbuf]
        ssems = sems[nbuf:]
        c = lax.axis_index("c")
        s = lax.axis_index("s")
        wid = c * NS + s
        row0 = pl.multiple_of(s * STRIPE, 8)
        pltpu.sync_copy(zeros_hbm, rows[0].at[pl.ds(0, ZK)])
        for j in range(STRIPE // ZK):
            pltpu.sync_copy(rows[0].at[pl.ds(0, ZK)],
                            acc_sh.at[pl.ds(row0 + j * ZK, ZK)])
        pltpu.sync_copy(src_hbm.at[wid], src_v)
        pltpu.sync_copy(dst_hbm.at[wid], dst_v)
        plsc.subcore_barrier()

        def g_start(i, q):
            pltpu.async_copy(h_hbm.at[src_v.at[i]], rows[q], gsems[q])

        def g_wait(i, q):
            pltpu.make_async_copy(h_hbm.at[src_v.at[i]], rows[q],
                                  gsems[q]).wait()

        def s_start(i, q):
            pltpu.async_copy(rows[q], acc_sh.at[dst_v.at[i]], ssems[q],
                             add=True)

        def s_wait(i, q):
            pltpu.make_async_copy(rows[q], acc_sh.at[dst_v.at[i]],
                                  ssems[q]).wait()

        for q in range(nbuf):
            g_start(q, q)

        def body(j, carry):
            i0 = nbuf * j
            for q in range(nbuf):
                g_wait(i0 + q, q)
                s_start(i0 + q, q)
            for q in range(nbuf):
                s_wait(i0 + q, q)
                nxt = i0 + nbuf + q

                @pl.when(nxt < NCHUNK)
                def _():
                    g_start(nxt, q)

            return carry

        lax.fori_loop(0, NCHUNK // nbuf, body, 0)
        for r in range(NCHUNK % nbuf):
            i = NCHUNK - (NCHUNK % nbuf) + r
            q = i % nbuf
            g_wait(i, q)
            s_start(i, q)
            s_wait(i, q)
        plsc.subcore_barrier()
        for j in range(STRIPE // ZK):
            pltpu.sync_copy(acc_sh.at[pl.ds(row0 + j * ZK, ZK)],
                            rows[0].at[pl.ds(0, ZK)])
            pltpu.sync_copy(rows[0].at[pl.ds(0, ZK)],
                            out_hbm.at[c, pl.ds(row0 + j * ZK, ZK)])

    return agg


_agg128 = _make_agg_kernel(NFEAT, 2)
_agg64 = _make_agg_kernel(NHID, 4)


_BLK = 1024
_GRID = N_PAD // _BLK


def _prep_body(feat_ref, dsrc_ref, ddst_ref, h1_ref, sout_ref, sin_ref):
    i = pl.program_id(0)
    d_out = jnp.sum(dsrc_ref[:, pl.ds(i * _BLK, _BLK)], axis=0)
    d_in = jnp.sum(ddst_ref[:, pl.ds(i * _BLK, _BLK)], axis=0)
    s_out = lax.rsqrt(jnp.maximum(d_out, 1.0))
    s_in = lax.rsqrt(jnp.maximum(d_in, 1.0))
    h1_ref[...] = feat_ref[...] * s_out[:, None]
    sout_ref[...] = s_out[:, None]
    sin_ref[...] = s_in[:, None]


_prep = pl.pallas_call(
    _prep_body,
    grid=(_GRID,),
    in_specs=[
        pl.BlockSpec((_BLK, NFEAT), lambda i: (i, 0)),
        pl.BlockSpec((NW, N_PAD), lambda i: (0, 0)),
        pl.BlockSpec((NW, N_PAD), lambda i: (0, 0)),
    ],
    out_specs=[
        pl.BlockSpec((_BLK, NFEAT), lambda i: (i, 0)),
        pl.BlockSpec((_BLK, 1), lambda i: (i, 0)),
        pl.BlockSpec((_BLK, 1), lambda i: (i, 0)),
    ],
    out_shape=[
        jax.ShapeDtypeStruct((N_PAD, NFEAT), jnp.float32),
        jax.ShapeDtypeStruct((N_PAD, 1), jnp.float32),
        jax.ShapeDtypeStruct((N_PAD, 1), jnp.float32),
    ],
)


def _mid_body(agg_ref, sin_ref, sout_ref, w1_ref, b1_ref, w2_ref, h2_ref):
    a = (agg_ref[0] + agg_ref[1]) * sin_ref[...]
    x = jnp.maximum(
        jnp.dot(a, w1_ref[...], preferred_element_type=jnp.float32)
        + b1_ref[...], 0.0)
    h2_ref[...] = jnp.dot(x * sout_ref[...], w2_ref[...],
                          preferred_element_type=jnp.float32)


_mid = pl.pallas_call(
    _mid_body,
    grid=(_GRID,),
    in_specs=[
        pl.BlockSpec((NC, _BLK, NFEAT), lambda i: (0, i, 0)),
        pl.BlockSpec((_BLK, 1), lambda i: (i, 0)),
        pl.BlockSpec((_BLK, 1), lambda i: (i, 0)),
        pl.BlockSpec((NFEAT, NFEAT), lambda i: (0, 0)),
        pl.BlockSpec((1, NFEAT), lambda i: (0, 0)),
        pl.BlockSpec((NFEAT, NHID), lambda i: (0, 0)),
    ],
    out_specs=pl.BlockSpec((_BLK, NHID), lambda i: (i, 0)),
    out_shape=jax.ShapeDtypeStruct((N_PAD, NHID), jnp.float32),
)


def _final_body(agg_ref, sin_ref, b2_ref, out_ref):
    z = (agg_ref[0] + agg_ref[1]) * sin_ref[...] + b2_ref[...]
    m = jnp.max(z, axis=1, keepdims=True)
    e = jnp.exp(z - m)
    out_ref[...] = (z - m) - jnp.log(jnp.sum(e, axis=1, keepdims=True))


_final = pl.pallas_call(
    _final_body,
    grid=(_GRID,),
    in_specs=[
        pl.BlockSpec((NC, _BLK, NHID), lambda i: (0, i, 0)),
        pl.BlockSpec((_BLK, 1), lambda i: (i, 0)),
        pl.BlockSpec((1, NHID), lambda i: (0, 0)),
    ],
    out_specs=pl.BlockSpec((_BLK, NHID), lambda i: (i, 0)),
    out_shape=jax.ShapeDtypeStruct((N_NODES, NHID), jnp.float32),
)


def kernel(feat, edge_index, W1, b1, W2, b2):
    src = edge_index[0].astype(jnp.int32)
    dst = edge_index[1].astype(jnp.int32)
    src3 = src.reshape(NW, NCHUNK, K)
    dst3 = dst.reshape(NW, NCHUNK, K)
    zeros_n = jnp.zeros((N_PAD,), jnp.float32)
    zeros128 = jnp.zeros((ZK, NFEAT), jnp.float32)
    zeros64 = jnp.zeros((ZK, NHID), jnp.float32)

    dsrc, ddst = _deg_kernel(src.reshape(NW, EPT), dst.reshape(NW, EPT),
                             zeros_n)
    h1, s_out, s_in = _prep(feat, dsrc, ddst)
    agg1 = _agg128(h1, src3, dst3, zeros128)
    h2 = _mid(agg1, s_in, s_out, W1, b1.reshape(1, NFEAT), W2)
    agg2 = _agg64(h2, src3, dst3, zeros64)
    return _final(agg2, s_in, b2.reshape(1, NHID))


# speedup vs baseline: 1.2145x; 1.1044x over previous
import functools

import jax
import jax.numpy as jnp
from jax import lax
from jax.experimental import pallas as pl
from jax.experimental.pallas import tpu as pltpu
from jax.experimental.pallas import tpu_sc as plsc

N_NODES = 10000
N_PAD = 10240
N_EDGES = 320000
NFEAT = 128
NHID = 64

NC = 2
NS = 16
NW = NC * NS
K = 80
NCHUNK = 125
EPT = NCHUNK * K
E_PAD = NW * EPT
N_ACC = N_NODES
STRIPE = N_ACC // NS

_mesh = plsc.VectorSubcoreMesh(core_axis_name="c", subcore_axis_name="s")


@functools.partial(
    pl.kernel,
    out_type=(
        jax.ShapeDtypeStruct((NW, N_PAD), jnp.float32),
        jax.ShapeDtypeStruct((NW, N_PAD), jnp.float32),
    ),
    mesh=_mesh,
    compiler_params=pltpu.CompilerParams(needs_layout_passes=False),
    scratch_types=(
        pltpu.VMEM((EPT,), jnp.int32),
        pltpu.VMEM((EPT,), jnp.int32),
        pltpu.VMEM((N_PAD,), jnp.float32),
        pltpu.VMEM((N_PAD,), jnp.float32),
    ),
)
def _deg_kernel(src_hbm, dst_hbm, zeros_hbm, dsrc_hbm, ddst_hbm,
                src_v, dst_v, hsrc_v, hdst_v):
    c = lax.axis_index("c")
    s = lax.axis_index("s")
    wid = c * NS + s
    pltpu.sync_copy(zeros_hbm, hsrc_v)
    pltpu.sync_copy(zeros_hbm, hdst_v)
    pltpu.sync_copy(src_hbm.at[wid], src_v)
    pltpu.sync_copy(dst_hbm.at[wid], dst_v)
    ones = jnp.full((16,), 1.0, jnp.float32)

    def body(i, carry):
        plsc.addupdate_scatter(hsrc_v, [src_v[pl.ds(i * 16, 16)]], ones)
        plsc.addupdate_scatter(hdst_v, [dst_v[pl.ds(i * 16, 16)]], ones)
        return carry

    lax.fori_loop(0, EPT // 16, body, 0)
    pltpu.sync_copy(hsrc_v, dsrc_hbm.at[wid])
    pltpu.sync_copy(hdst_v, ddst_hbm.at[wid])


def _make_agg_kernel(d, nbuf):
    @functools.partial(
        pl.kernel,
        out_type=jax.ShapeDtypeStruct((NC, N_ACC, d), jnp.float32),
        mesh=_mesh,
        compiler_params=pltpu.CompilerParams(use_tc_tiling_on_sc=False),
        scratch_types=(
            pltpu.VMEM((NCHUNK, K), jnp.int32),
            pltpu.VMEM((NCHUNK, K), jnp.int32),
            tuple(pltpu.VMEM((K, d), jnp.float32) for _ in range(nbuf)),
            pltpu.VMEM_SHARED((N_ACC, d), jnp.float32),
            tuple(pltpu.SemaphoreType.DMA for _ in range(2 * nbuf)),
        ),
    )
    def agg(h_hbm, src_hbm, dst_hbm, zeros_hbm, out_hbm,
            src_v, dst_v, rows, acc_sh, sems):
        gsems = sems[:nbuf]
        ssems = sems[nbuf:]
        c = lax.axis_index("c")
        s = lax.axis_index("s")
        wid = c * NS + s
        row0 = s * STRIPE
        pltpu.sync_copy(zeros_hbm, acc_sh.at[pl.ds(row0, STRIPE)])
        pltpu.sync_copy(src_hbm.at[wid], src_v)
        pltpu.sync_copy(dst_hbm.at[wid], dst_v)
        plsc.subcore_barrier()

        def g_start(i, q):
            pltpu.async_copy(h_hbm.at[src_v.at[i]], rows[q], gsems[q])

        def g_wait(i, q):
            pltpu.make_async_copy(h_hbm.at[src_v.at[i]], rows[q],
                                  gsems[q]).wait()

        def s_start(i, q):
            pltpu.async_copy(rows[q], acc_sh.at[dst_v.at[i]], ssems[q],
                             add=True)

        def s_wait(i, q):
            pltpu.make_async_copy(rows[q], acc_sh.at[dst_v.at[i]],
                                  ssems[q]).wait()

        for q in range(nbuf):
            g_start(q, q)

        def body(j, carry):
            i0 = nbuf * j
            for q in range(nbuf):
                g_wait(i0 + q, q)
                s_start(i0 + q, q)
            for q in range(nbuf):
                s_wait(i0 + q, q)
                nxt = i0 + nbuf + q

                @pl.when(nxt < NCHUNK)
                def _():
                    g_start(nxt, q)

            return carry

        lax.fori_loop(0, NCHUNK // nbuf, body, 0)
        for r in range(NCHUNK % nbuf):
            i = NCHUNK - (NCHUNK % nbuf) + r
            q = i % nbuf
            g_wait(i, q)
            s_start(i, q)
            s_wait(i, q)
        plsc.subcore_barrier()
        pltpu.sync_copy(acc_sh.at[pl.ds(row0, STRIPE)],
                        out_hbm.at[c, pl.ds(row0, STRIPE)])

    return agg


_agg128 = _make_agg_kernel(NFEAT, 3)
_agg64 = _make_agg_kernel(NHID, 4)


_BLK = 1024
_GRID = N_PAD // _BLK


def _prep_body(feat_ref, dsrc_ref, ddst_ref, h1_ref, sout_ref, sin_ref):
    i = pl.program_id(0)
    d_out = jnp.sum(dsrc_ref[:, pl.ds(i * _BLK, _BLK)], axis=0)
    d_in = jnp.sum(ddst_ref[:, pl.ds(i * _BLK, _BLK)], axis=0)
    s_out = lax.rsqrt(jnp.maximum(d_out, 1.0))
    s_in = lax.rsqrt(jnp.maximum(d_in, 1.0))
    h1_ref[...] = feat_ref[...] * s_out[:, None]
    sout_ref[...] = s_out[:, None]
    sin_ref[...] = s_in[:, None]


_prep = pl.pallas_call(
    _prep_body,
    grid=(_GRID,),
    in_specs=[
        pl.BlockSpec((_BLK, NFEAT), lambda i: (i, 0)),
        pl.BlockSpec((NW, N_PAD), lambda i: (0, 0)),
        pl.BlockSpec((NW, N_PAD), lambda i: (0, 0)),
    ],
    out_specs=[
        pl.BlockSpec((_BLK, NFEAT), lambda i: (i, 0)),
        pl.BlockSpec((_BLK, 1), lambda i: (i, 0)),
        pl.BlockSpec((_BLK, 1), lambda i: (i, 0)),
    ],
    out_shape=[
        jax.ShapeDtypeStruct((N_PAD, NFEAT), jnp.float32),
        jax.ShapeDtypeStruct((N_PAD, 1), jnp.float32),
        jax.ShapeDtypeStruct((N_PAD, 1), jnp.float32),
    ],
)


def _mid_body(agg_ref, sin_ref, sout_ref, w1_ref, b1_ref, w2_ref, h2_ref):
    a = (agg_ref[0] + agg_ref[1]) * sin_ref[...]
    x = jnp.maximum(
        jnp.dot(a, w1_ref[...], preferred_element_type=jnp.float32)
        + b1_ref[...], 0.0)
    h2_ref[...] = jnp.dot(x * sout_ref[...], w2_ref[...],
                          preferred_element_type=jnp.float32)


_mid = pl.pallas_call(
    _mid_body,
    grid=(_GRID,),
    in_specs=[
        pl.BlockSpec((NC, _BLK, NFEAT), lambda i: (0, i, 0)),
        pl.BlockSpec((_BLK, 1), lambda i: (i, 0)),
        pl.BlockSpec((_BLK, 1), lambda i: (i, 0)),
        pl.BlockSpec((NFEAT, NFEAT), lambda i: (0, 0)),
        pl.BlockSpec((1, NFEAT), lambda i: (0, 0)),
        pl.BlockSpec((NFEAT, NHID), lambda i: (0, 0)),
    ],
    out_specs=pl.BlockSpec((_BLK, NHID), lambda i: (i, 0)),
    out_shape=jax.ShapeDtypeStruct((N_PAD, NHID), jnp.float32),
)


def _final_body(agg_ref, sin_ref, b2_ref, out_ref):
    z = (agg_ref[0] + agg_ref[1]) * sin_ref[...] + b2_ref[...]
    m = jnp.max(z, axis=1, keepdims=True)
    e = jnp.exp(z - m)
    out_ref[...] = (z - m) - jnp.log(jnp.sum(e, axis=1, keepdims=True))


_final = pl.pallas_call(
    _final_body,
    grid=(_GRID,),
    in_specs=[
        pl.BlockSpec((NC, _BLK, NHID), lambda i: (0, i, 0)),
        pl.BlockSpec((_BLK, 1), lambda i: (i, 0)),
        pl.BlockSpec((1, NHID), lambda i: (0, 0)),
    ],
    out_specs=pl.BlockSpec((_BLK, NHID), lambda i: (i, 0)),
    out_shape=jax.ShapeDtypeStruct((N_NODES, NHID), jnp.float32),
)


def kernel(feat, edge_index, W1, b1, W2, b2):
    src = edge_index[0].astype(jnp.int32)
    dst = edge_index[1].astype(jnp.int32)
    src3 = src.reshape(NW, NCHUNK, K)
    dst3 = dst.reshape(NW, NCHUNK, K)
    zeros_n = jnp.zeros((N_PAD,), jnp.float32)
    zeros128 = jnp.zeros((STRIPE, NFEAT), jnp.float32)
    zeros64 = jnp.zeros((STRIPE, NHID), jnp.float32)

    dsrc, ddst = _deg_kernel(src.reshape(NW, EPT), dst.reshape(NW, EPT),
                             zeros_n)
    h1, s_out, s_in = _prep(feat, dsrc, ddst)
    agg1 = _agg128(h1, src3, dst3, zeros128)
    h2 = _mid(agg1, s_in, s_out, W1, b1.reshape(1, NFEAT), W2)
    agg2 = _agg64(h2, src3, dst3, zeros64)
    return _final(agg2, s_in, b2.reshape(1, NHID))


# speedup vs baseline: 1.2247x; 1.0084x over previous
import functools

import jax
import jax.numpy as jnp
from jax import lax
from jax.experimental import pallas as pl
from jax.experimental.pallas import tpu as pltpu
from jax.experimental.pallas import tpu_sc as plsc

N_NODES = 10000
N_PAD = 10240
N_EDGES = 320000
NFEAT = 128
NHID = 64

NC = 2
NS = 16
NW = NC * NS
K = 80
NCHUNK = 125
EPT = NCHUNK * K
E_PAD = NW * EPT
N_ACC = N_NODES
STRIPE = N_ACC // NS

_mesh = plsc.VectorSubcoreMesh(core_axis_name="c", subcore_axis_name="s")


@functools.partial(
    pl.kernel,
    out_type=(
        jax.ShapeDtypeStruct((NW, N_PAD), jnp.float32),
        jax.ShapeDtypeStruct((NW, N_PAD), jnp.float32),
    ),
    mesh=_mesh,
    compiler_params=pltpu.CompilerParams(needs_layout_passes=False),
    scratch_types=(
        pltpu.VMEM((EPT,), jnp.int32),
        pltpu.VMEM((EPT,), jnp.int32),
        pltpu.VMEM((N_PAD,), jnp.float32),
        pltpu.VMEM((N_PAD,), jnp.float32),
    ),
)
def _deg_kernel(src_hbm, dst_hbm, zeros_hbm, dsrc_hbm, ddst_hbm,
                src_v, dst_v, hsrc_v, hdst_v):
    c = lax.axis_index("c")
    s = lax.axis_index("s")
    wid = c * NS + s
    pltpu.sync_copy(zeros_hbm, hsrc_v)
    pltpu.sync_copy(zeros_hbm, hdst_v)
    pltpu.sync_copy(src_hbm.at[wid], src_v)
    pltpu.sync_copy(dst_hbm.at[wid], dst_v)
    ones = jnp.full((16,), 1.0, jnp.float32)

    def body(i, carry):
        plsc.addupdate_scatter(hsrc_v, [src_v[pl.ds(i * 16, 16)]], ones)
        plsc.addupdate_scatter(hdst_v, [dst_v[pl.ds(i * 16, 16)]], ones)
        return carry

    lax.fori_loop(0, EPT // 16, body, 0)
    pltpu.sync_copy(hsrc_v, dsrc_hbm.at[wid])
    pltpu.sync_copy(hdst_v, ddst_hbm.at[wid])


def _make_agg_kernel(d, nbuf):
    @functools.partial(
        pl.kernel,
        out_type=jax.ShapeDtypeStruct((NC, N_ACC, d), jnp.float32),
        mesh=_mesh,
        compiler_params=pltpu.CompilerParams(use_tc_tiling_on_sc=False),
        scratch_types=(
            pltpu.VMEM((NCHUNK, K), jnp.int32),
            pltpu.VMEM((NCHUNK, K), jnp.int32),
            tuple(pltpu.VMEM((K, d), jnp.float32) for _ in range(nbuf)),
            pltpu.VMEM_SHARED((N_ACC, d), jnp.float32),
            tuple(pltpu.SemaphoreType.DMA for _ in range(2 * nbuf)),
        ),
    )
    def agg(h_hbm, src_hbm, dst_hbm, zeros_hbm, out_hbm,
            src_v, dst_v, rows, acc_sh, sems):
        gsems = sems[:nbuf]
        ssems = sems[nbuf:]
        c = lax.axis_index("c")
        s = lax.axis_index("s")
        wid = c * NS + s
        row0 = s * STRIPE
        pltpu.sync_copy(zeros_hbm, acc_sh.at[pl.ds(row0, STRIPE)])
        pltpu.sync_copy(src_hbm.at[wid], src_v)
        pltpu.sync_copy(dst_hbm.at[wid], dst_v)
        plsc.subcore_barrier()

        def g_start(i, q):
            pltpu.async_copy(h_hbm.at[src_v.at[i]], rows[q], gsems[q])

        def g_wait(i, q):
            pltpu.make_async_copy(h_hbm.at[src_v.at[i]], rows[q],
                                  gsems[q]).wait()

        def s_start(i, q):
            pltpu.async_copy(rows[q], acc_sh.at[dst_v.at[i]], ssems[q],
                             add=True)

        def s_wait(i, q):
            pltpu.make_async_copy(rows[q], acc_sh.at[dst_v.at[i]],
                                  ssems[q]).wait()

        for q in range(nbuf):
            g_start(q, q)

        def body(j, carry):
            i0 = nbuf * j
            for q in range(nbuf):
                g_wait(i0 + q, q)
                s_start(i0 + q, q)
            for q in range(nbuf):
                s_wait(i0 + q, q)
                nxt = i0 + nbuf + q

                @pl.when(nxt < NCHUNK)
                def _():
                    g_start(nxt, q)

            return carry

        lax.fori_loop(0, NCHUNK // nbuf, body, 0)
        for r in range(NCHUNK % nbuf):
            i = NCHUNK - (NCHUNK % nbuf) + r
            q = i % nbuf
            g_wait(i, q)
            s_start(i, q)
            s_wait(i, q)
        plsc.subcore_barrier()
        pltpu.sync_copy(acc_sh.at[pl.ds(row0, STRIPE)],
                        out_hbm.at[c, pl.ds(row0, STRIPE)])

    return agg


_agg128 = _make_agg_kernel(NFEAT, 3)
_agg64 = _make_agg_kernel(NHID, 6)


_BLK = 1024
_GRID = N_PAD // _BLK


def _prep_body(feat_ref, dsrc_ref, ddst_ref, h1_ref, sout_ref, sin_ref):
    i = pl.program_id(0)
    d_out = jnp.sum(dsrc_ref[:, pl.ds(i * _BLK, _BLK)], axis=0)
    d_in = jnp.sum(ddst_ref[:, pl.ds(i * _BLK, _BLK)], axis=0)
    s_out = lax.rsqrt(jnp.maximum(d_out, 1.0))
    s_in = lax.rsqrt(jnp.maximum(d_in, 1.0))
    h1_ref[...] = feat_ref[...] * s_out[:, None]
    sout_ref[...] = s_out[:, None]
    sin_ref[...] = s_in[:, None]


_prep = pl.pallas_call(
    _prep_body,
    grid=(_GRID,),
    in_specs=[
        pl.BlockSpec((_BLK, NFEAT), lambda i: (i, 0)),
        pl.BlockSpec((NW, N_PAD), lambda i: (0, 0)),
        pl.BlockSpec((NW, N_PAD), lambda i: (0, 0)),
    ],
    out_specs=[
        pl.BlockSpec((_BLK, NFEAT), lambda i: (i, 0)),
        pl.BlockSpec((_BLK, 1), lambda i: (i, 0)),
        pl.BlockSpec((_BLK, 1), lambda i: (i, 0)),
    ],
    out_shape=[
        jax.ShapeDtypeStruct((N_PAD, NFEAT), jnp.float32),
        jax.ShapeDtypeStruct((N_PAD, 1), jnp.float32),
        jax.ShapeDtypeStruct((N_PAD, 1), jnp.float32),
    ],
)


def _mid_body(agg_ref, sin_ref, sout_ref, w1_ref, b1_ref, w2_ref, h2_ref):
    a = (agg_ref[0] + agg_ref[1]) * sin_ref[...]
    x = jnp.maximum(
        jnp.dot(a, w1_ref[...], preferred_element_type=jnp.float32)
        + b1_ref[...], 0.0)
    h2_ref[...] = jnp.dot(x * sout_ref[...], w2_ref[...],
                          preferred_element_type=jnp.float32)


_mid = pl.pallas_call(
    _mid_body,
    grid=(_GRID,),
    in_specs=[
        pl.BlockSpec((NC, _BLK, NFEAT), lambda i: (0, i, 0)),
        pl.BlockSpec((_BLK, 1), lambda i: (i, 0)),
        pl.BlockSpec((_BLK, 1), lambda i: (i, 0)),
        pl.BlockSpec((NFEAT, NFEAT), lambda i: (0, 0)),
        pl.BlockSpec((1, NFEAT), lambda i: (0, 0)),
        pl.BlockSpec((NFEAT, NHID), lambda i: (0, 0)),
    ],
    out_specs=pl.BlockSpec((_BLK, NHID), lambda i: (i, 0)),
    out_shape=jax.ShapeDtypeStruct((N_PAD, NHID), jnp.float32),
)


def _final_body(agg_ref, sin_ref, b2_ref, out_ref):
    z = (agg_ref[0] + agg_ref[1]) * sin_ref[...] + b2_ref[...]
    m = jnp.max(z, axis=1, keepdims=True)
    e = jnp.exp(z - m)
    out_ref[...] = (z - m) - jnp.log(jnp.sum(e, axis=1, keepdims=True))


_final = pl.pallas_call(
    _final_body,
    grid=(_GRID,),
    in_specs=[
        pl.BlockSpec((NC, _BLK, NHID), lambda i: (0, i, 0)),
        pl.BlockSpec((_BLK, 1), lambda i: (i, 0)),
        pl.BlockSpec((1, NHID), lambda i: (0, 0)),
    ],
    out_specs=pl.BlockSpec((_BLK, NHID), lambda i: (i, 0)),
    out_shape=jax.ShapeDtypeStruct((N_NODES, NHID), jnp.float32),
)


def kernel(feat, edge_index, W1, b1, W2, b2):
    src = edge_index[0].astype(jnp.int32)
    dst = edge_index[1].astype(jnp.int32)
    src3 = src.reshape(NW, NCHUNK, K)
    dst3 = dst.reshape(NW, NCHUNK, K)
    zeros_n = jnp.zeros((N_PAD,), jnp.float32)
    zeros128 = jnp.zeros((STRIPE, NFEAT), jnp.float32)
    zeros64 = jnp.zeros((STRIPE, NHID), jnp.float32)

    dsrc, ddst = _deg_kernel(src.reshape(NW, EPT), dst.reshape(NW, EPT),
                             zeros_n)
    h1, s_out, s_in = _prep(feat, dsrc, ddst)
    agg1 = _agg128(h1, src3, dst3, zeros128)
    h2 = _mid(agg1, s_in, s_out, W1, b1.reshape(1, NFEAT), W2)
    agg2 = _agg64(h2, src3, dst3, zeros64)
    return _final(agg2, s_in, b2.reshape(1, NHID))


# speedup vs baseline: 1.2276x; 1.0024x over previous
import functools

import jax
import jax.numpy as jnp
from jax import lax
from jax.experimental import pallas as pl
from jax.experimental.pallas import tpu as pltpu
from jax.experimental.pallas import tpu_sc as plsc

N_NODES = 10000
N_PAD = 10240
N_EDGES = 320000
NFEAT = 128
NHID = 64

NC = 2
NS = 16
NW = NC * NS
K = 80
NCHUNK = 125
EPT = NCHUNK * K
N_ACC = N_NODES
STRIPE = N_ACC // NS

_mesh = plsc.VectorSubcoreMesh(core_axis_name="c", subcore_axis_name="s")


@functools.partial(
    pl.kernel,
    out_type=(
        jax.ShapeDtypeStruct((NW, N_PAD), jnp.float32),
        jax.ShapeDtypeStruct((NW, N_PAD), jnp.float32),
    ),
    mesh=_mesh,
    compiler_params=pltpu.CompilerParams(needs_layout_passes=False),
    scratch_types=(
        pltpu.VMEM((EPT,), jnp.int32),
        pltpu.VMEM((EPT,), jnp.int32),
        pltpu.VMEM((N_PAD,), jnp.float32),
        pltpu.VMEM((N_PAD,), jnp.float32),
    ),
)
def _deg_kernel(src_hbm, dst_hbm, zeros_hbm, dsrc_hbm, ddst_hbm,
                src_v, dst_v, hsrc_v, hdst_v):
    c = lax.axis_index("c")
    s = lax.axis_index("s")
    wid = c * NS + s
    pltpu.sync_copy(zeros_hbm, hsrc_v)
    pltpu.sync_copy(zeros_hbm, hdst_v)
    pltpu.sync_copy(src_hbm.at[wid], src_v)
    pltpu.sync_copy(dst_hbm.at[wid], dst_v)
    ones = jnp.full((16,), 1.0, jnp.float32)

    def body(i, carry):
        plsc.addupdate_scatter(hsrc_v, [src_v[pl.ds(i * 16, 16)]], ones)
        plsc.addupdate_scatter(hdst_v, [dst_v[pl.ds(i * 16, 16)]], ones)
        return carry

    lax.fori_loop(0, EPT // 16, body, 0)
    pltpu.sync_copy(hsrc_v, dsrc_hbm.at[wid])
    pltpu.sync_copy(hdst_v, ddst_hbm.at[wid])


def _make_agg_kernel(d, nbuf):
    @functools.partial(
        pl.kernel,
        out_type=jax.ShapeDtypeStruct((NC, N_ACC, d), jnp.float32),
        mesh=_mesh,
        compiler_params=pltpu.CompilerParams(use_tc_tiling_on_sc=False),
        scratch_types=(
            pltpu.VMEM((NCHUNK, K), jnp.int32),
            pltpu.VMEM((NCHUNK, K), jnp.int32),
            tuple(pltpu.VMEM((K, d), jnp.float32) for _ in range(nbuf)),
            pltpu.VMEM_SHARED((N_ACC, d), jnp.float32),
            tuple(pltpu.SemaphoreType.DMA for _ in range(2 * nbuf)),
        ),
    )
    def agg(h_hbm, src_hbm, dst_hbm, zeros_hbm, out_hbm,
            src_v, dst_v, rows, acc_sh, sems):
        gsems = sems[:nbuf]
        ssems = sems[nbuf:]
        c = lax.axis_index("c")
        s = lax.axis_index("s")
        wid = c * NS + s
        row0 = s * STRIPE
        pltpu.sync_copy(zeros_hbm, acc_sh.at[pl.ds(row0, STRIPE)])
        pltpu.sync_copy(src_hbm.at[wid], src_v)
        pltpu.sync_copy(dst_hbm.at[wid], dst_v)
        plsc.subcore_barrier()

        def g_start(i, q):
            pltpu.async_copy(h_hbm.at[src_v.at[i]], rows[q], gsems[q])

        def g_wait(i, q):
            pltpu.make_async_copy(h_hbm.at[src_v.at[i]], rows[q],
                                  gsems[q]).wait()

        def s_start(i, q):
            pltpu.async_copy(rows[q], acc_sh.at[dst_v.at[i]], ssems[q],
                             add=True)

        def s_wait(i, q):
            pltpu.make_async_copy(rows[q], acc_sh.at[dst_v.at[i]],
                                  ssems[q]).wait()

        for q in range(nbuf):
            g_start(q, q)

        def body(j, carry):
            i0 = nbuf * j
            for q in range(nbuf):
                g_wait(i0 + q, q)
                s_start(i0 + q, q)
            for q in range(nbuf):
                s_wait(i0 + q, q)
                nxt = i0 + nbuf + q

                @pl.when(nxt < NCHUNK)
                def _():
                    g_start(nxt, q)

            return carry

        lax.fori_loop(0, NCHUNK // nbuf, body, 0)
        for r in range(NCHUNK % nbuf):
            i = NCHUNK - (NCHUNK % nbuf) + r
            q = i % nbuf
            g_wait(i, q)
            s_start(i, q)
            s_wait(i, q)
        plsc.subcore_barrier()
        pltpu.sync_copy(acc_sh.at[pl.ds(row0, STRIPE)],
                        out_hbm.at[c, pl.ds(row0, STRIPE)])

    return agg


_agg128 = _make_agg_kernel(NFEAT, 3)
_agg64 = _make_agg_kernel(NHID, 6)


_BLK = 1024
_GRID = N_PAD // _BLK


def _prep_body(feat_ref, dsrc_ref, ddst_ref, h1_ref, sout_ref, sin_ref):
    i = pl.program_id(0)
    d_out = jnp.sum(dsrc_ref[:, pl.ds(i * _BLK, _BLK)], axis=0)
    d_in = jnp.sum(ddst_ref[:, pl.ds(i * _BLK, _BLK)], axis=0)
    s_out = lax.rsqrt(jnp.maximum(d_out, 1.0))
    s_in = lax.rsqrt(jnp.maximum(d_in, 1.0))
    h1_ref[...] = feat_ref[...] * s_out[:, None]
    sout_ref[...] = s_out[:, None]
    sin_ref[...] = s_in[:, None]


_prep = pl.pallas_call(
    _prep_body,
    grid=(_GRID,),
    in_specs=[
        pl.BlockSpec((_BLK, NFEAT), lambda i: (i, 0)),
        pl.BlockSpec((NW, N_PAD), lambda i: (0, 0)),
        pl.BlockSpec((NW, N_PAD), lambda i: (0, 0)),
    ],
    out_specs=[
        pl.BlockSpec((_BLK, NFEAT), lambda i: (i, 0)),
        pl.BlockSpec((_BLK, 1), lambda i: (i, 0)),
        pl.BlockSpec((_BLK, 1), lambda i: (i, 0)),
    ],
    out_shape=[
        jax.ShapeDtypeStruct((N_PAD, NFEAT), jnp.float32),
        jax.ShapeDtypeStruct((N_PAD, 1), jnp.float32),
        jax.ShapeDtypeStruct((N_PAD, 1), jnp.float32),
    ],
)


def _mid_body(agg_ref, sin_ref, sout_ref, w1_ref, b1_ref, w2_ref, h2_ref):
    a = (agg_ref[0] + agg_ref[1]) * sin_ref[...]
    x = jnp.maximum(
        jnp.dot(a, w1_ref[...], preferred_element_type=jnp.float32)
        + b1_ref[...], 0.0)
    h2_ref[...] = jnp.dot(x * sout_ref[...], w2_ref[...],
                          preferred_element_type=jnp.float32)


_mid = pl.pallas_call(
    _mid_body,
    grid=(_GRID,),
    in_specs=[
        pl.BlockSpec((NC, _BLK, NFEAT), lambda i: (0, i, 0)),
        pl.BlockSpec((_BLK, 1), lambda i: (i, 0)),
        pl.BlockSpec((_BLK, 1), lambda i: (i, 0)),
        pl.BlockSpec((NFEAT, NFEAT), lambda i: (0, 0)),
        pl.BlockSpec((1, NFEAT), lambda i: (0, 0)),
        pl.BlockSpec((NFEAT, NHID), lambda i: (0, 0)),
    ],
    out_specs=pl.BlockSpec((_BLK, NHID), lambda i: (i, 0)),
    out_shape=jax.ShapeDtypeStruct((N_PAD, NHID), jnp.float32),
)


def _final_body(agg_ref, sin_ref, b2_ref, out_ref):
    z = (agg_ref[0] + agg_ref[1]) * sin_ref[...] + b2_ref[...]
    m = jnp.max(z, axis=1, keepdims=True)
    e = jnp.exp(z - m)
    out_ref[...] = (z - m) - jnp.log(jnp.sum(e, axis=1, keepdims=True))


_final = pl.pallas_call(
    _final_body,
    grid=(_GRID,),
    in_specs=[
        pl.BlockSpec((NC, _BLK, NHID), lambda i: (0, i, 0)),
        pl.BlockSpec((_BLK, 1), lambda i: (i, 0)),
        pl.BlockSpec((1, NHID), lambda i: (0, 0)),
    ],
    out_specs=pl.BlockSpec((_BLK, NHID), lambda i: (i, 0)),
    out_shape=jax.ShapeDtypeStruct((N_NODES, NHID), jnp.float32),
)


def kernel(feat, edge_index, W1, b1, W2, b2):
    src = edge_index[0].astype(jnp.int32)
    dst = edge_index[1].astype(jnp.int32)
    src3 = src.reshape(NW, NCHUNK, K)
    dst3 = dst.reshape(NW, NCHUNK, K)
    zeros_n = jnp.zeros((N_PAD,), jnp.float32)
    zeros128 = jnp.zeros((STRIPE, NFEAT), jnp.float32)
    zeros64 = jnp.zeros((STRIPE, NHID), jnp.float32)

    dsrc, ddst = _deg_kernel(src.reshape(NW, EPT), dst.reshape(NW, EPT),
                             zeros_n)
    h1, s_out, s_in = _prep(feat, dsrc, ddst)
    agg1 = _agg128(h1, src3, dst3, zeros128)
    h2 = _mid(agg1, s_in, s_out, W1, b1.reshape(1, NFEAT), W2)
    agg2 = _agg64(h2, src3, dst3, zeros64)
    return _final(agg2, s_in, b2.reshape(1, NHID))
